# baseline (speedup 1.0000x reference)
import functools

import jax
import jax.numpy as jnp
from jax.experimental import pallas as pl
from jax.experimental.pallas import tpu as pltpu

_BF = jnp.bfloat16


def _tile_rows(a, n):
    return jnp.concatenate([a] * n, axis=0)


def _tile_lanes(a, n):
    return jnp.concatenate([a] * n, axis=1)


def _band(tab, kdim, cin, cout, win, wout, shift):
    r = jax.lax.broadcasted_iota(jnp.int32, (win * cin, wout * cout), 0)
    q = jax.lax.broadcasted_iota(jnp.int32, (win * cin, wout * cout), 1)
    w, u = r // cin, q // cout
    acc = None
    for k in range(kdim):
        t = _tile_lanes(_tile_rows(tab[k].astype(_BF), win), wout)
        v = jnp.where(w == 2 * u + k + shift, t, jnp.zeros_like(t))
        acc = v if acc is None else acc + v
    return acc


def _fused_kernel(x_ref, w1m_ref, b1_ref, w2m_ref, b2_ref, w3m_ref, b3_ref,
                  fc1_ref, fb1_ref, fc2_ref, fb2_ref, wh_ref, bh_ref,
                  val_ref, adv_ref, *, bb, C, H1, W1, H2, W2, PH, PW,
                  c1, c2, c3, n_act):
    def mm(a, b):
        return jnp.dot(a, b, preferred_element_type=jnp.float32)

    w1m = w1m_ref[...].reshape(4, 4, C, c1)
    r1 = jax.lax.broadcasted_iota(jnp.int32, (4 * W1, W1 * c1), 0)
    q1 = jax.lax.broadcasted_iota(jnp.int32, (4 * W1, W1 * c1), 1)
    m1 = r1 // 4 == q1 // c1
    s1 = [[jnp.where(m1, _tile_lanes(_tile_rows(
        w1m[ki, :, c, :].astype(_BF), W1), W1), 0).astype(_BF)
        for ki in range(4)] for c in range(C)]
    w2m = w2m_ref[...].reshape(2, 2, c1, c2)
    w3m = w3m_ref[...].reshape(2, 2, c2, c3)
    s2 = [_band(w2m[ki], 2, c1, c2, W1, W2, 0) for ki in range(2)]
    s3a = [_band(w3m[ki], 2, c2, c3, W2, PW, 0) for ki in range(2)]
    s3b = [_band(w3m[ki], 2, c2, c3, W2, PW, 1) for ki in range(2)]

    xb = x_ref[...].astype(_BF).reshape(bb, C, H1, 4, 4 * W1)
    acc = None
    for c in range(C):
        for ki in range(4):
            xs = xb[:, c, :, ki, :].reshape(bb * H1, 4 * W1)
            t = mm(xs, s1[c][ki])
            acc = t if acc is None else acc + t
    y1 = jnp.maximum(acc + _tile_lanes(b1_ref[...], W1), 0.0).astype(_BF)

    y1 = y1.reshape(bb, H1, W1 * c1)[:, :2 * H2, :].reshape(bb, H2, 2, W1 * c1)
    acc = None
    for ki in range(2):
        xs = y1[:, :, ki, :].reshape(bb * H2, W1 * c1)
        t = mm(xs, s2[ki])
        acc = t if acc is None else acc + t
    y2 = jnp.maximum(acc + _tile_lanes(b2_ref[...], W2), 0.0).astype(_BF)

    y2 = y2.reshape(bb, H2, W2 * c2)
    ya = None
    yb = None
    for ki in range(2):
        xs = y2[:, ki:ki + 2 * PH, :].reshape(bb * 2 * PH, W2 * c2)
        ta = mm(xs, s3a[ki])
        tb = mm(xs, s3b[ki])
        ya = ta if ya is None else ya + ta
        yb = tb if yb is None else yb + tb
    z = jnp.maximum(ya, yb).reshape(bb, PH, 2, PW * c3)
    pooled = (jnp.maximum(z[:, :, 0, :], z[:, :, 1, :])
              + _tile_lanes(b3_ref[...], PW)).astype(_BF)

    k = PW * c3
    w1 = fc1_ref[...].astype(_BF)
    h = mm(pooled[:, 0, :], w1[0:k, :])
    for ph in range(1, PH):
        h = h + mm(pooled[:, ph, :], w1[ph * k:(ph + 1) * k, :])
    h = jnp.maximum(h + fb1_ref[...], 0.0).astype(_BF)
    h = jnp.maximum(mm(h, fc2_ref[...].astype(_BF)) + fb2_ref[...],
                    0.0).astype(_BF)
    res = mm(h, wh_ref[...].astype(_BF)) + bh_ref[...]
    val_ref[...] = res[:, 0:1].astype(val_ref.dtype)
    adv_ref[...] = res[:, 1:1 + n_act].astype(adv_ref.dtype)


def kernel(conv1_wm, conv1_b, conv2_wm, conv2_b, conv3_wm, conv3_b,
           fc1_wm, fc1_b, fc2_wm, fc2_b, head_wm, head_b, x):
    B, C, H, W = x.shape
    H1, W1 = H // 4, W // 4
    H2, W2 = H1 // 2, W1 // 2
    PH, PW = (H2 - 1) // 2, (W2 - 1) // 2
    c1 = conv1_wm.shape[1]
    c2 = conv2_wm.shape[1]
    c3 = conv3_wm.shape[1]

    bb = 32
    while B % bb:
        bb //= 2

    def rep(arr):
        s = arr.shape
        return pl.BlockSpec(s, lambda i: (0,) * len(s))

    ws = [conv1_wm, conv1_b, conv2_wm, conv2_b, conv3_wm, conv3_b,
          fc1_wm, fc1_b, fc2_wm, fc2_b, head_wm, head_b]

    n_act = 12
    value, advantage = pl.pallas_call(
        functools.partial(_fused_kernel, bb=bb, C=C, H1=H1, W1=W1, H2=H2,
                          W2=W2, PH=PH, PW=PW, c1=c1, c2=c2, c3=c3,
                          n_act=n_act),
        out_shape=[jax.ShapeDtypeStruct((B, 1), jnp.float32),
                   jax.ShapeDtypeStruct((B, n_act), jnp.float32)],
        grid=(B // bb,),
        in_specs=[pl.BlockSpec((bb, C, H, W), lambda i: (i, 0, 0, 0))]
        + [rep(w) for w in ws],
        out_specs=[pl.BlockSpec((bb, 1), lambda i: (i, 0)),
                   pl.BlockSpec((bb, n_act), lambda i: (i, 0))],
        compiler_params=pltpu.CompilerParams(
            dimension_semantics=("parallel",),
            vmem_limit_bytes=100 * 1024 * 1024),
    )(x, *ws)

    return value, advantage


# speedup vs baseline: 1.0095x; 1.0095x over previous
import functools

import jax
import jax.numpy as jnp
from jax.experimental import pallas as pl
from jax.experimental.pallas import tpu as pltpu

_BF = jnp.bfloat16


def _tile_rows(a, n):
    return jnp.concatenate([a] * n, axis=0)


def _tile_lanes(a, n):
    return jnp.concatenate([a] * n, axis=1)


def _band(tab, kdim, cin, cout, win, wout, shift):
    r = jax.lax.broadcasted_iota(jnp.int32, (win * cin, wout * cout), 0)
    q = jax.lax.broadcasted_iota(jnp.int32, (win * cin, wout * cout), 1)
    w, u = r // cin, q // cout
    acc = None
    for k in range(kdim):
        t = _tile_lanes(_tile_rows(tab[k].astype(_BF), win), wout)
        v = jnp.where(w == 2 * u + k + shift, t, jnp.zeros_like(t))
        acc = v if acc is None else acc + v
    return acc


def _fused_kernel(x_ref, w1m_ref, b1_ref, w2m_ref, b2_ref, w3m_ref, b3_ref,
                  fc1_ref, fb1_ref, fc2_ref, fb2_ref, wh_ref, bh_ref,
                  val_ref, adv_ref, *, bb, C, H1, W1, H2, W2, PH, PW,
                  c1, c2, c3, n_act):
    def mm(a, b):
        return jnp.dot(a, b, preferred_element_type=jnp.float32)

    w1m = w1m_ref[...].reshape(4, 4, C, c1)
    r1 = jax.lax.broadcasted_iota(jnp.int32, (4 * W1, W1 * c1), 0)
    q1 = jax.lax.broadcasted_iota(jnp.int32, (4 * W1, W1 * c1), 1)
    m1 = r1 // 4 == q1 // c1
    s1 = [[jnp.where(m1, _tile_lanes(_tile_rows(
        w1m[ki, :, c, :].astype(_BF), W1), W1), 0).astype(_BF)
        for ki in range(4)] for c in range(C)]
    w2m = w2m_ref[...].reshape(2, 2, c1, c2)
    w3m = w3m_ref[...].reshape(2, 2, c2, c3)
    s2 = [_band(w2m[ki], 2, c1, c2, W1, W2, 0) for ki in range(2)]
    s3a = [_band(w3m[ki], 2, c2, c3, W2, PW, 0) for ki in range(2)]
    s3b = [_band(w3m[ki], 2, c2, c3, W2, PW, 1) for ki in range(2)]

    xb = x_ref[...].astype(_BF).reshape(bb, C, H1, 4, 4 * W1)
    acc = None
    for c in range(C):
        for ki in range(4):
            xs = xb[:, c, :, ki, :].reshape(bb * H1, 4 * W1)
            t = mm(xs, s1[c][ki])
            acc = t if acc is None else acc + t
    y1 = jnp.maximum(acc + _tile_lanes(b1_ref[...], W1), 0.0).astype(_BF)

    y1 = y1.reshape(bb, H1, W1 * c1)[:, :2 * H2, :].reshape(bb, H2, 2, W1 * c1)
    acc = None
    for ki in range(2):
        xs = y1[:, :, ki, :].reshape(bb * H2, W1 * c1)
        t = mm(xs, s2[ki])
        acc = t if acc is None else acc + t
    y2 = jnp.maximum(acc + _tile_lanes(b2_ref[...], W2), 0.0).astype(_BF)

    y2 = y2.reshape(bb, H2, W2 * c2)
    ya = None
    yb = None
    for ki in range(2):
        xs = y2[:, ki:ki + 2 * PH, :].reshape(bb * 2 * PH, W2 * c2)
        ta = mm(xs, s3a[ki])
        tb = mm(xs, s3b[ki])
        ya = ta if ya is None else ya + ta
        yb = tb if yb is None else yb + tb
    z = jnp.maximum(ya, yb).reshape(bb, PH, 2, PW * c3)
    pooled = (jnp.maximum(z[:, :, 0, :], z[:, :, 1, :])
              + _tile_lanes(b3_ref[...], PW)).astype(_BF)

    k = PW * c3
    w1 = fc1_ref[...].astype(_BF)
    h = mm(pooled[:, 0, :], w1[0:k, :])
    for ph in range(1, PH):
        h = h + mm(pooled[:, ph, :], w1[ph * k:(ph + 1) * k, :])
    h = jnp.maximum(h + fb1_ref[...], 0.0).astype(_BF)
    h = jnp.maximum(mm(h, fc2_ref[...].astype(_BF)) + fb2_ref[...],
                    0.0).astype(_BF)
    res = mm(h, wh_ref[...].astype(_BF)) + bh_ref[...]
    val_ref[...] = res[:, 0:1].astype(val_ref.dtype)
    adv_ref[...] = res[:, 1:1 + n_act].astype(adv_ref.dtype)


def kernel(conv1_wm, conv1_b, conv2_wm, conv2_b, conv3_wm, conv3_b,
           fc1_wm, fc1_b, fc2_wm, fc2_b, head_wm, head_b, x):
    B, C, H, W = x.shape
    H1, W1 = H // 4, W // 4
    H2, W2 = H1 // 2, W1 // 2
    PH, PW = (H2 - 1) // 2, (W2 - 1) // 2
    c1 = conv1_wm.shape[1]
    c2 = conv2_wm.shape[1]
    c3 = conv3_wm.shape[1]

    bb = 64
    while B % bb:
        bb //= 2

    def rep(arr):
        s = arr.shape
        return pl.BlockSpec(s, lambda i: (0,) * len(s))

    ws = [conv1_wm, conv1_b, conv2_wm, conv2_b, conv3_wm, conv3_b,
          fc1_wm, fc1_b, fc2_wm, fc2_b, head_wm, head_b]

    n_act = 12
    value, advantage = pl.pallas_call(
        functools.partial(_fused_kernel, bb=bb, C=C, H1=H1, W1=W1, H2=H2,
                          W2=W2, PH=PH, PW=PW, c1=c1, c2=c2, c3=c3,
                          n_act=n_act),
        out_shape=[jax.ShapeDtypeStruct((B, 1), jnp.float32),
                   jax.ShapeDtypeStruct((B, n_act), jnp.float32)],
        grid=(B // bb,),
        in_specs=[pl.BlockSpec((bb, C, H, W), lambda i: (i, 0, 0, 0))]
        + [rep(w) for w in ws],
        out_specs=[pl.BlockSpec((bb, 1), lambda i: (i, 0)),
                   pl.BlockSpec((bb, n_act), lambda i: (i, 0))],
        compiler_params=pltpu.CompilerParams(
            dimension_semantics=("parallel",),
            vmem_limit_bytes=100 * 1024 * 1024),
    )(x, *ws)

    return value, advantage


# speedup vs baseline: 1.0898x; 1.0796x over previous
import functools

import jax
import jax.numpy as jnp
from jax.experimental import pallas as pl
from jax.experimental.pallas import tpu as pltpu

_BF = jnp.bfloat16


def _tile_rows(a, n):
    return jnp.concatenate([a] * n, axis=0)


def _tile_lanes(a, n):
    return jnp.concatenate([a] * n, axis=1)


def _band(tab, kdim, cin, cout, win, wout, shift):
    r = jax.lax.broadcasted_iota(jnp.int32, (win * cin, wout * cout), 0)
    q = jax.lax.broadcasted_iota(jnp.int32, (win * cin, wout * cout), 1)
    w, u = r // cin, q // cout
    acc = None
    for k in range(kdim):
        t = _tile_lanes(_tile_rows(tab[k].astype(_BF), win), wout)
        v = jnp.where(w == 2 * u + k + shift, t, jnp.zeros_like(t))
        acc = v if acc is None else acc + v
    return acc


def _fused_kernel(x_ref,
                  w1m_ref, b1_ref, w2m_ref, b2_ref, w3m_ref, b3_ref,
                  fc1_ref, fb1_ref, fc2_ref, fb2_ref, wh_ref, bh_ref,
                  val_ref, adv_ref, *, bb, C, H1, W1, H2, W2, PH, PW,
                  c1, c2, c3, n_act):
    def mm(a, b):
        return jnp.dot(a, b, preferred_element_type=jnp.float32)

    w1m = w1m_ref[...].reshape(4, 4, C, c1)
    r1 = jax.lax.broadcasted_iota(jnp.int32, (4 * W1, W1 * c1), 0)
    q1 = jax.lax.broadcasted_iota(jnp.int32, (4 * W1, W1 * c1), 1)
    m1 = r1 // 4 == q1 // c1
    s1 = [jnp.concatenate(
        [jnp.where(m1, _tile_lanes(_tile_rows(
            w1m[ki, :, c, :].astype(_BF), W1), W1), 0).astype(_BF)
         for c in range(C)], axis=0)
        for ki in range(4)]
    w2m = w2m_ref[...].reshape(2, 2, c1, c2)
    w3m = w3m_ref[...].reshape(2, 2, c2, c3)
    s2 = [_band(w2m[ki], 2, c1, c2, W1, W2, 0) for ki in range(2)]
    s3a = [_band(w3m[ki], 2, c2, c3, W2, PW, 0) for ki in range(2)]
    s3b = [_band(w3m[ki], 2, c2, c3, W2, PW, 1) for ki in range(2)]

    xb = x_ref[...].astype(_BF).reshape(bb, C, H1, 4, 4 * W1)
    acc = None
    for ki in range(4):
        v = xb[:, :, :, ki, :]
        xs = jnp.concatenate([v[:, c] for c in range(C)], axis=-1)
        t = mm(xs.reshape(bb * H1, C * 4 * W1), s1[ki])
        acc = t if acc is None else acc + t
    y1 = jnp.maximum(acc + _tile_lanes(b1_ref[...], W1), 0.0).astype(_BF)

    y1 = y1.reshape(bb, H1, W1 * c1)[:, :2 * H2, :].reshape(bb, H2, 2, W1 * c1)
    acc = None
    for ki in range(2):
        xs = y1[:, :, ki, :].reshape(bb * H2, W1 * c1)
        t = mm(xs, s2[ki])
        acc = t if acc is None else acc + t
    y2 = jnp.maximum(acc + _tile_lanes(b2_ref[...], W2), 0.0).astype(_BF)

    y2 = y2.reshape(bb, H2, W2 * c2)
    ya = None
    yb = None
    for ki in range(2):
        xs = y2[:, ki:ki + 2 * PH, :].reshape(bb * 2 * PH, W2 * c2)
        ta = mm(xs, s3a[ki])
        tb = mm(xs, s3b[ki])
        ya = ta if ya is None else ya + ta
        yb = tb if yb is None else yb + tb
    z = jnp.maximum(ya, yb).reshape(bb, PH, 2, PW * c3)
    pooled = (jnp.maximum(z[:, :, 0, :], z[:, :, 1, :])
              + _tile_lanes(b3_ref[...], PW)).astype(_BF)

    k = PW * c3
    w1 = fc1_ref[...].astype(_BF)
    h = mm(pooled[:, 0, :], w1[0:k, :])
    for ph in range(1, PH):
        h = h + mm(pooled[:, ph, :], w1[ph * k:(ph + 1) * k, :])
    h = jnp.maximum(h + fb1_ref[...], 0.0).astype(_BF)
    h = jnp.maximum(mm(h, fc2_ref[...].astype(_BF)) + fb2_ref[...],
                    0.0).astype(_BF)
    res = mm(h, wh_ref[...].astype(_BF)) + bh_ref[...]
    val_ref[...] = res[:, 0:1].astype(val_ref.dtype)
    adv_ref[...] = res[:, 1:1 + n_act].astype(adv_ref.dtype)


def kernel(conv1_wm, conv1_b, conv2_wm, conv2_b, conv3_wm, conv3_b,
           fc1_wm, fc1_b, fc2_wm, fc2_b, head_wm, head_b, x):
    B, C, H, W = x.shape
    H1, W1 = H // 4, W // 4
    H2, W2 = H1 // 2, W1 // 2
    PH, PW = (H2 - 1) // 2, (W2 - 1) // 2
    c1 = conv1_wm.shape[1]
    c2 = conv2_wm.shape[1]
    c3 = conv3_wm.shape[1]

    bb = 64
    while B % bb:
        bb //= 2

    def rep(arr):
        s = arr.shape
        return pl.BlockSpec(s, lambda i: (0,) * len(s))

    ws = [conv1_wm, conv1_b, conv2_wm, conv2_b, conv3_wm, conv3_b,
          fc1_wm, fc1_b, fc2_wm, fc2_b, head_wm, head_b]

    n_act = 12
    value, advantage = pl.pallas_call(
        functools.partial(_fused_kernel, bb=bb, C=C, H1=H1, W1=W1, H2=H2,
                          W2=W2, PH=PH, PW=PW, c1=c1, c2=c2, c3=c3,
                          n_act=n_act),
        out_shape=[jax.ShapeDtypeStruct((B, 1), jnp.float32),
                   jax.ShapeDtypeStruct((B, n_act), jnp.float32)],
        grid=(B // bb,),
        in_specs=[pl.BlockSpec((bb, C, H, W), lambda i: (i, 0, 0, 0))]
        + [rep(w) for w in ws],
        out_specs=[pl.BlockSpec((bb, 1), lambda i: (i, 0)),
                   pl.BlockSpec((bb, n_act), lambda i: (i, 0))],
        compiler_params=pltpu.CompilerParams(
            dimension_semantics=("parallel",),
            vmem_limit_bytes=100 * 1024 * 1024),
    )(x, *ws)

    return value, advantage
